# Initial kernel scaffold; baseline (speedup 1.0000x reference)
import jax
import jax.numpy as jnp
from jax.experimental import pallas as pl


def kernel(p3_cls, p3_reg, p3_mask, p3_map, p4_cls, p4_reg, p4_mask, p4_map, p5_cls, p5_reg, p5_mask, p5_map):
    raise NotImplementedError("write your pallas kernel here")


# speedup vs baseline: 4.5829x; 4.5829x over previous
import functools

import jax
import jax.numpy as jnp
from jax.experimental import pallas as pl
from jax.experimental.pallas import tpu as pltpu

_K = 8
_OHEM_RATIO = 3.0
_NEG_FILL = -1e30
_BISECT_ITERS = 40


def _level_kernel(cls_ref, msk_ref, rx_ref, ry_ref, gx_ref, gy_ref,
                  out_ref, acc_ref, negce_ref, *, nsteps, total):
    nc = pl.num_programs(1)
    step = pl.program_id(0) * nc + pl.program_id(1)

    @pl.when(step == 0)
    def _init():
        for i in range(8):
            acc_ref[i] = 0.0

    c = cls_ref[0].astype(jnp.float32)
    m = msk_ref[0].astype(jnp.float32)
    tr = m[0:1]
    tcl = m[1:2]
    tm = m[2:3]
    x1 = (c[0:1] - c[1:2]) * (2.0 * tr - 1.0)
    ce_tr = jnp.maximum(x1, 0.0) + jnp.log1p(jnp.exp(-jnp.abs(x1)))
    x2 = (c[2:3] - c[3:4]) * (2.0 * tcl - 1.0)
    ce_tcl = jnp.maximum(x2, 0.0) + jnp.log1p(jnp.exp(-jnp.abs(x2)))

    pos = tr * tm
    neg = (1.0 - tr) * tm
    negce = jnp.where(neg > 0.0, ce_tr, _NEG_FILL)
    negce_ref[step] = negce

    wm = pos * (tr + tcl) * 0.125
    dx = jnp.abs(gx_ref[0] - rx_ref[0])
    slx = jnp.where(dx < 1.0, 0.5 * dx * dx, dx - 0.5)
    dy = jnp.abs(gy_ref[0] - ry_ref[0])
    sly = jnp.where(dy < 1.0, 0.5 * dy * dy, dy - 0.5)

    acc_ref[0] += jnp.sum(pos)
    acc_ref[1] += jnp.sum(ce_tr * pos)
    acc_ref[2] += jnp.sum(neg)
    acc_ref[3] += jnp.sum(ce_tcl * pos)
    acc_ref[4] += jnp.sum(ce_tcl)
    acc_ref[5] += jnp.sum(slx * wm)
    acc_ref[6] += jnp.sum(sly * wm)
    acc_ref[7] = jnp.maximum(acc_ref[7],
                             jnp.max(jnp.where(neg > 0.0, ce_tr, 0.0)))

    @pl.when(step == nsteps - 1)
    def _finalize():
        n_pos = acc_ref[0]
        s_ce_pos = acc_ref[1]
        n_neg_all = acc_ref[2]
        s_tcl_pos = acc_ref[3]
        s_tcl_all = acc_ref[4]
        sx = acc_ref[5]
        sy = acc_ref[6]
        maxv = acc_ref[7]

        has_pos = n_pos > 0.0
        n_neg = jnp.where(has_pos,
                          jnp.minimum(n_neg_all,
                                      jnp.floor(_OHEM_RATIO * n_pos)),
                          100.0)
        eff = jnp.minimum(n_neg, n_neg_all)

        v = negce_ref[...]

        def body(_, carry):
            lo, hi = carry
            mid = 0.5 * (lo + hi)
            cnt = jnp.sum((v > mid).astype(jnp.float32))
            take_lo = cnt >= eff
            return (jnp.where(take_lo, mid, lo), jnp.where(take_lo, hi, mid))

        lo, hi = jax.lax.fori_loop(0, _BISECT_ITERS, body, (0.0, maxv))
        cnt_hi = jnp.sum((v > hi).astype(jnp.float32))
        sum_hi = jnp.sum(jnp.where(v > hi, v, 0.0))
        loss_neg = sum_hi + (eff - cnt_hi) * hi

        loss_pos = jnp.where(has_pos, s_ce_pos, 0.0)
        l_tr = (loss_pos + loss_neg) / (n_pos + n_neg)

        tcl_pos = s_tcl_pos / jnp.maximum(n_pos, 1.0)
        tcl_neg = (s_tcl_all - s_tcl_pos) / jnp.maximum(total - n_pos, 1.0)
        l_tcl = jnp.where(has_pos, tcl_pos + 0.5 * tcl_neg, 0.0)

        denom = jnp.maximum(n_pos * float(_K), 1.0)
        l_rx = jnp.where(has_pos, sx / denom, 0.0)
        l_ry = jnp.where(has_pos, sy / denom, 0.0)

        out_ref[0] = l_tr
        out_ref[1] = l_tcl
        out_ref[2] = l_rx
        out_ref[3] = l_ry


def _run_level(cls_a, reg_a, msk_a, map_a, interpret=False):
    n, _, s, _ = cls_a.shape
    S = s * s
    total = float(n * S)
    nc = max(1, S // 6400)
    B = S // nc
    nsteps = n * nc

    cls_r = cls_a.reshape(n, 4, S)
    msk_r = msk_a.reshape(n, 3, S)
    reg_r = reg_a.reshape(n, reg_a.shape[1], S)
    map_r = map_a.reshape(n, map_a.shape[1], S)

    kern = functools.partial(_level_kernel, nsteps=nsteps, total=total)
    return pl.pallas_call(
        kern,
        grid=(n, nc),
        in_specs=[
            pl.BlockSpec((1, 4, B), lambda i, j: (i, 0, j)),
            pl.BlockSpec((1, 3, B), lambda i, j: (i, 0, j)),
            pl.BlockSpec((1, _K, B), lambda i, j: (i, 0, j)),
            pl.BlockSpec((1, _K, B), lambda i, j: (i, 1, j)),
            pl.BlockSpec((1, _K, B), lambda i, j: (i, 0, j)),
            pl.BlockSpec((1, _K, B), lambda i, j: (i, 1, j)),
        ],
        out_specs=pl.BlockSpec(memory_space=pltpu.SMEM),
        out_shape=jax.ShapeDtypeStruct((4,), jnp.float32),
        scratch_shapes=[
            pltpu.SMEM((8,), jnp.float32),
            pltpu.VMEM((nsteps, 1, B), jnp.float32),
        ],
        interpret=interpret,
    )(cls_r, msk_r, reg_r, reg_r, map_r, map_r)


def kernel(p3_cls, p3_reg, p3_mask, p3_map,
           p4_cls, p4_reg, p4_mask, p4_map,
           p5_cls, p5_reg, p5_mask, p5_map):
    o3 = _run_level(p3_cls, p3_reg, p3_mask, p3_map)
    o4 = _run_level(p4_cls, p4_reg, p4_mask, p4_map)
    o5 = _run_level(p5_cls, p5_reg, p5_mask, p5_map)
    return o3 + o4 + o5


# speedup vs baseline: 8.4291x; 1.8392x over previous
import functools

import jax
import jax.numpy as jnp
from jax.experimental import pallas as pl
from jax.experimental.pallas import tpu as pltpu

_K = 8
_OHEM_RATIO = 3.0
_NEG_FILL = -1e30
_BISECT_ITERS = 40


def _softplus(x):
    return jnp.maximum(x, 0.0) + jnp.log1p(jnp.exp(-jnp.abs(x)))


def _level_kernel(cls_ref, msk_ref, rx_ref, ry_ref, gx_ref, gy_ref,
                  out_ref, accm_ref, accx_ref, accy_ref, maxa_ref,
                  negce_ref, sel_ref, *, nsteps, total):
    nc = pl.num_programs(1)
    step = pl.program_id(0) * nc + pl.program_id(1)

    @pl.when(step == 0)
    def _init():
        accm_ref[...] = jnp.zeros_like(accm_ref)
        accx_ref[...] = jnp.zeros_like(accx_ref)
        accy_ref[...] = jnp.zeros_like(accy_ref)
        maxa_ref[...] = jnp.zeros_like(maxa_ref)

    c = cls_ref[0]
    m = msk_ref[0].astype(jnp.float32)
    tr = m[0]
    tcl = m[1]
    tm = m[2]
    ce_tr = _softplus((c[0] - c[1]) * (2.0 * tr - 1.0))
    ce_tcl = _softplus((c[2] - c[3]) * (2.0 * tcl - 1.0))

    pos = tr * tm
    neg = tm - pos
    negce = jnp.where(neg > 0.0, ce_tr, _NEG_FILL)
    negce_ref[step] = negce

    accm_ref[0] += pos
    accm_ref[1] += ce_tr * pos
    accm_ref[2] += neg
    accm_ref[3] += ce_tcl * pos
    accm_ref[4] += ce_tcl
    accm_ref[5] += ce_tr * neg
    maxa_ref[...] = jnp.maximum(maxa_ref[...], jnp.maximum(negce, 0.0))

    wm = (pos * (tr + tcl) * 0.125)[None]
    dx = jnp.abs(gx_ref[0] - rx_ref[0])
    mx = jnp.minimum(dx, 1.0)
    accx_ref[...] += (dx - mx + 0.5 * mx * mx) * wm
    dy = jnp.abs(gy_ref[0] - ry_ref[0])
    my = jnp.minimum(dy, 1.0)
    accy_ref[...] += (dy - my + 0.5 * my * my) * wm

    @pl.when(step == nsteps - 1)
    def _finalize():
        n_pos = jnp.sum(accm_ref[0])
        s_ce_pos = jnp.sum(accm_ref[1])
        n_neg_all = jnp.sum(accm_ref[2])
        s_tcl_pos = jnp.sum(accm_ref[3])
        s_tcl_all = jnp.sum(accm_ref[4])
        s_neg_all = jnp.sum(accm_ref[5])
        sx = jnp.sum(accx_ref[...])
        sy = jnp.sum(accy_ref[...])
        maxv = jnp.max(maxa_ref[...])

        has_pos = n_pos > 0.0
        n_neg = jnp.where(has_pos,
                          jnp.minimum(n_neg_all,
                                      jnp.floor(_OHEM_RATIO * n_pos)),
                          100.0)
        eff = jnp.minimum(n_neg, n_neg_all)
        need_select = eff < n_neg_all

        @pl.when(need_select)
        def _bisect():
            v = negce_ref[...]

            def body(_, carry):
                lo, hi = carry
                mid = 0.5 * (lo + hi)
                cnt = jnp.sum((v > mid).astype(jnp.float32))
                take_lo = cnt >= eff
                return (jnp.where(take_lo, mid, lo),
                        jnp.where(take_lo, hi, mid))

            lo, hi = jax.lax.fori_loop(0, _BISECT_ITERS, body, (0.0, maxv))
            cnt_hi = jnp.sum((v > hi).astype(jnp.float32))
            sum_hi = jnp.sum(jnp.where(v > hi, v, 0.0))
            sel_ref[0] = sum_hi + (eff - cnt_hi) * hi

        loss_neg = jnp.where(need_select, sel_ref[0], s_neg_all)
        loss_pos = jnp.where(has_pos, s_ce_pos, 0.0)
        l_tr = (loss_pos + loss_neg) / (n_pos + n_neg)

        tcl_pos = s_tcl_pos / jnp.maximum(n_pos, 1.0)
        tcl_neg = (s_tcl_all - s_tcl_pos) / jnp.maximum(total - n_pos, 1.0)
        l_tcl = jnp.where(has_pos, tcl_pos + 0.5 * tcl_neg, 0.0)

        denom = jnp.maximum(n_pos * float(_K), 1.0)
        l_rx = jnp.where(has_pos, sx / denom, 0.0)
        l_ry = jnp.where(has_pos, sy / denom, 0.0)

        out_ref[0] = l_tr
        out_ref[1] = l_tcl
        out_ref[2] = l_rx
        out_ref[3] = l_ry


def _run_level(cls_a, reg_a, msk_a, map_a, nc, interpret=False):
    n, _, s, _ = cls_a.shape
    S = s * s
    S8 = S // 8
    total = float(n * S)
    L = S8 // nc
    nsteps = n * nc

    cls_r = cls_a.reshape(n, 4, 8, S8)
    msk_r = msk_a.reshape(n, 3, 8, S8)
    reg_r = reg_a.reshape(n, reg_a.shape[1], 8, S8)
    map_r = map_a.reshape(n, map_a.shape[1], 8, S8)

    kern = functools.partial(_level_kernel, nsteps=nsteps, total=total)
    return pl.pallas_call(
        kern,
        grid=(n, nc),
        in_specs=[
            pl.BlockSpec((1, 4, 8, L), lambda i, j: (i, 0, 0, j)),
            pl.BlockSpec((1, 3, 8, L), lambda i, j: (i, 0, 0, j)),
            pl.BlockSpec((1, _K, 8, L), lambda i, j: (i, 0, 0, j)),
            pl.BlockSpec((1, _K, 8, L), lambda i, j: (i, 1, 0, j)),
            pl.BlockSpec((1, _K, 8, L), lambda i, j: (i, 0, 0, j)),
            pl.BlockSpec((1, _K, 8, L), lambda i, j: (i, 1, 0, j)),
        ],
        out_specs=pl.BlockSpec(memory_space=pltpu.SMEM),
        out_shape=jax.ShapeDtypeStruct((4,), jnp.float32),
        scratch_shapes=[
            pltpu.VMEM((6, 8, L), jnp.float32),
            pltpu.VMEM((_K, 8, L), jnp.float32),
            pltpu.VMEM((_K, 8, L), jnp.float32),
            pltpu.VMEM((8, L), jnp.float32),
            pltpu.VMEM((nsteps, 8, L), jnp.float32),
            pltpu.SMEM((1,), jnp.float32),
        ],
        interpret=interpret,
    )(cls_r, msk_r, reg_r, reg_r, map_r, map_r)


def kernel(p3_cls, p3_reg, p3_mask, p3_map,
           p4_cls, p4_reg, p4_mask, p4_map,
           p5_cls, p5_reg, p5_mask, p5_map):
    o3 = _run_level(p3_cls, p3_reg, p3_mask, p3_map, nc=5)
    o4 = _run_level(p4_cls, p4_reg, p4_mask, p4_map, nc=1)
    o5 = _run_level(p5_cls, p5_reg, p5_mask, p5_map, nc=1)
    return o3 + o4 + o5
